# baseline (speedup 1.0000x reference)
import jax
import jax.numpy as jnp
from jax import lax
from jax.experimental import pallas as pl
from jax.experimental.pallas import tpu as pltpu
from jax.experimental.pallas import tpu_sc as plsc

_D = 64
_SEQ = 200
_NC = 2
_NS = 16
_NW = _NC * _NS
_BPW = 128
_NV = _D // 16
_V = 1000000
_NCHUNK = _V // 128
_TAIL = _V - _NCHUNK * 128
_SLOTS = -(-_NCHUNK // _NW)


def _conv_body(tableT_hbm, tail_hbm, outf_hbm, vb0, vb1, wb, ob0, ob1,
               sem0, sem1):
    wid = lax.axis_index("s") * _NC + lax.axis_index("c")

    iota = lax.iota(jnp.int32, 16)

    def fire(j, vb, sem):
        cid = wid + _NW * j

        @pl.when(cid < _NCHUNK)
        def _():
            pltpu.async_copy(tableT_hbm.at[:, pl.ds(cid * 128, 128)],
                             vb, sem)

    def drain(j, vb, sem):
        cid = wid + _NW * j

        @pl.when(cid < _NCHUNK)
        def _():
            pltpu.make_async_copy(tableT_hbm.at[:, pl.ds(0, 128)],
                                  vb, sem).wait()

    def work(j, vb, ob):
        cid = wid + _NW * j

        @pl.when(cid < _NCHUNK)
        def _():
            @plsc.parallel_loop(0, _D, unroll=2)
            def _(d):
                dv = jnp.broadcast_to(d, (16,))
                for k in range(128 // 16):
                    v = vb[d, pl.ds(k * 16, 16)]
                    plsc.store_scatter(wb, [iota + 16 * k, dv], v)

            @plsc.parallel_loop(0, 128, unroll=4)
            def _(b):
                for s in range(_NV):
                    ob[pl.ds(b * _D + s * 16, 16)] = wb[b, pl.ds(s * 16, 16)]
            pltpu.sync_copy(ob, outf_hbm.at[pl.ds(cid * 128 * _D, 128 * _D)])

    @pl.when(wid == 0)
    def _():
        pltpu.sync_copy(tail_hbm, outf_hbm.at[pl.ds(_NCHUNK * 128 * _D,
                                                    _TAIL * _D)])

    fire(0, vb0, sem0)

    def loop_body(t, _):
        j = 2 * t
        fire(j + 1, vb1, sem1)
        drain(j, vb0, sem0)
        work(j, vb0, ob0)

        @pl.when(j + 2 < _SLOTS)
        def _():
            fire(j + 2, vb0, sem0)
        drain(j + 1, vb1, sem1)
        work(j + 1, vb1, ob1)
        return ()

    lax.fori_loop(0, (_SLOTS + 1) // 2, loop_body, ())


def _sc_body(table_hbm, xt_hbm, pe_hbm, out_hbm,
             idx_v, pe_v, rows0, rows1, out0, out1, gsem0, gsem1):
    wid = lax.axis_index("s") * _NC + lax.axis_index("c")
    b0 = wid * _BPW

    pltpu.sync_copy(xt_hbm.at[:, pl.ds(b0, _BPW)], idx_v)
    pltpu.sync_copy(pe_hbm, pe_v)

    def fire(l, rows_b, gsem):
        pltpu.async_copy(table_hbm.at[idx_v.at[l]], rows_b, gsem)

    def wait_gather(rows_b, gsem):
        pltpu.make_async_copy(table_hbm.at[pl.ds(0, _BPW)], rows_b,
                              gsem).wait()

    iota = lax.iota(jnp.int32, 16)
    half = lax.shift_right_logical(iota, 1 + 1 + 1)
    r_sub = [half + 2 * s for s in range(_NV)]
    dr_vec = lax.bitwise_and(iota, 7)

    def compute(l, rows_b, out_b):
        pvec = [pe_v[l, pl.ds(s * 16, 16)] for s in range(_NV)]

        @plsc.parallel_loop(0, _BPW, unroll=4)
        def _(b):
            bc = jnp.broadcast_to(b, (16,))
            for s in range(_NV):
                v = rows_b[b, pl.ds(s * 16, 16)] * 8.0 + pvec[s]
                plsc.store_scatter(out_b, [r_sub[s], dr_vec, bc], v)

    def store(l, out_b):
        pltpu.sync_copy(out_b.at[:, :, pl.ds(0, 128)], out_hbm.at[l, :, wid])

    fire(0, rows0, gsem0)

    def loop_body(t, _):
        l = 2 * t
        fire(l + 1, rows1, gsem1)
        wait_gather(rows0, gsem0)
        compute(l, rows0, out0)
        store(l, out0)

        @pl.when(l + 2 < _SEQ)
        def _():
            fire(l + 2, rows0, gsem0)
        wait_gather(rows1, gsem1)
        compute(l + 1, rows1, out1)
        store(l + 1, out1)
        return ()

    lax.fori_loop(0, _SEQ // 2, loop_body, ())


def kernel(x, table, pe):
    b, seq = x.shape
    assert seq == _SEQ and b == _NW * _BPW and table.shape[0] == _V
    mesh = plsc.VectorSubcoreMesh(core_axis_name="c", subcore_axis_name="s",
                                  num_cores=_NC, num_subcores=_NS)

    tableT = table.T
    tail = table[_NCHUNK * 128:].reshape(-1)
    conv_kernel = pl.kernel(
        _conv_body,
        out_type=jax.ShapeDtypeStruct((_V * _D,), jnp.float32),
        mesh=mesh,
        scratch_types=[
            pltpu.VMEM((_D, 128), jnp.float32),
            pltpu.VMEM((_D, 128), jnp.float32),
            pltpu.VMEM((128, 65), jnp.float32),
            pltpu.VMEM((128 * _D,), jnp.float32),
            pltpu.VMEM((128 * _D,), jnp.float32),
            pltpu.SemaphoreType.DMA,
            pltpu.SemaphoreType.DMA,
        ],
        compiler_params=pltpu.CompilerParams(use_tc_tiling_on_sc=True,
                                             needs_layout_passes=False),
    )
    table_rm = conv_kernel(tableT, tail).reshape(_V, _D)

    xt = x.T.astype(jnp.int32)
    pe2 = pe[0, :seq, :]
    gather_kernel = pl.kernel(
        _sc_body,
        out_type=jax.ShapeDtypeStruct((_SEQ, _D // 8, b // 128, 8, 128),
                                      jnp.float32),
        mesh=mesh,
        scratch_types=[
            pltpu.VMEM((_SEQ, _BPW), jnp.int32),
            pltpu.VMEM((_SEQ, _D), jnp.float32),
            pltpu.VMEM((_BPW, _D), jnp.float32),
            pltpu.VMEM((_BPW, _D), jnp.float32),
            pltpu.VMEM((_D // 8, 8, 129), jnp.float32),
            pltpu.VMEM((_D // 8, 8, 129), jnp.float32),
            pltpu.SemaphoreType.DMA,
            pltpu.SemaphoreType.DMA,
        ],
        compiler_params=pltpu.CompilerParams(use_tc_tiling_on_sc=False,
                                             needs_layout_passes=False),
    )
    out5 = gather_kernel(table_rm, xt, pe2)
    return out5.transpose(2, 4, 0, 1, 3).reshape(b, seq, _D)


# speedup vs baseline: 1.5461x; 1.5461x over previous
import functools
import jax
import jax.numpy as jnp
from jax import lax
from jax.experimental import pallas as pl
from jax.experimental.pallas import tpu as pltpu
from jax.experimental.pallas import tpu_sc as plsc

_D = 64
_SEQ = 200
_NC = 2
_NS = 16
_NW = _NC * _NS
_BPW = 128
_NV = _D // 16


def _sc_body(table_hbm, xt_hbm, pe_hbm, out_hbm,
             idx_v, pe_v, rows0, rows1, out0, out1, gsem0, gsem1,
             ssem0, ssem1):
    wid = lax.axis_index("s") * _NC + lax.axis_index("c")
    b0 = wid * _BPW

    pltpu.sync_copy(xt_hbm.at[:, pl.ds(b0, _BPW)], idx_v)
    pltpu.sync_copy(pe_hbm, pe_v)

    def fire(l, rows_b, gsem):
        pltpu.async_copy(table_hbm.at[idx_v.at[l]], rows_b, gsem)

    def wait_gather(rows_b, gsem):
        pltpu.make_async_copy(table_hbm.at[pl.ds(0, _BPW)], rows_b,
                              gsem).wait()

    iota = lax.iota(jnp.int32, 16)
    half = lax.shift_right_logical(iota, 1 + 1 + 1)
    r_sub = [half + 2 * s for s in range(_NV)]
    dr_vec = lax.bitwise_and(iota, 7)

    def compute(l, rows_b, out_b):
        pvec = [pe_v[l, pl.ds(s * 16, 16)] for s in range(_NV)]

        @plsc.parallel_loop(0, _BPW, unroll=4)
        def _(b):
            bc = jnp.broadcast_to(b, (16,))
            for s in range(_NV):
                v = rows_b[b, pl.ds(s * 16, 16)] * 8.0 + pvec[s]
                plsc.store_scatter(out_b, [r_sub[s], dr_vec, bc], v)

    def store(l, out_b, ssem):
        pltpu.async_copy(out_b.at[:, :, pl.ds(0, 128)], out_hbm.at[l, :, wid],
                         ssem)

    def wait_store(out_b, ssem):
        pltpu.make_async_copy(out_b.at[:, :, pl.ds(0, 128)],
                              out_hbm.at[0, :, wid], ssem).wait()

    fire(0, rows0, gsem0)

    def loop_body(t, _):
        l = 2 * t
        fire(l + 1, rows1, gsem1)
        wait_gather(rows0, gsem0)

        @pl.when(t > 0)
        def _():
            wait_store(out0, ssem0)
        compute(l, rows0, out0)
        store(l, out0, ssem0)

        @pl.when(l + 2 < _SEQ)
        def _():
            fire(l + 2, rows0, gsem0)
        wait_gather(rows1, gsem1)

        @pl.when(t > 0)
        def _():
            wait_store(out1, ssem1)
        compute(l + 1, rows1, out1)
        store(l + 1, out1, ssem1)
        return ()

    lax.fori_loop(0, _SEQ // 2, loop_body, ())
    wait_store(out0, ssem0)
    wait_store(out1, ssem1)


def kernel(x, table, pe):
    b, seq = x.shape
    assert seq == _SEQ and b == _NW * _BPW
    xt = x.T.astype(jnp.int32)
    pe2 = pe[0, :seq, :]

    mesh = plsc.VectorSubcoreMesh(core_axis_name="c", subcore_axis_name="s",
                                  num_cores=_NC, num_subcores=_NS)
    grid_kernel = pl.kernel(
        _sc_body,
        out_type=jax.ShapeDtypeStruct((_SEQ, _D // 8, b // 128, 8, 128),
                                      jnp.float32),
        mesh=mesh,
        scratch_types=[
            pltpu.VMEM((_SEQ, _BPW), jnp.int32),
            pltpu.VMEM((_SEQ, _D), jnp.float32),
            pltpu.VMEM((_BPW, _D), jnp.float32),
            pltpu.VMEM((_BPW, _D), jnp.float32),
            pltpu.VMEM((_D // 8, 8, 129), jnp.float32),
            pltpu.VMEM((_D // 8, 8, 129), jnp.float32),
            pltpu.SemaphoreType.DMA,
            pltpu.SemaphoreType.DMA,
            pltpu.SemaphoreType.DMA,
            pltpu.SemaphoreType.DMA,
        ],
        compiler_params=pltpu.CompilerParams(use_tc_tiling_on_sc=False,
                                             needs_layout_passes=False),
    )
    out5 = grid_kernel(table, xt, pe2)
    return out5.transpose(2, 4, 0, 1, 3).reshape(b, seq, _D)


# speedup vs baseline: 1.5462x; 1.0001x over previous
import jax
import jax.numpy as jnp
from jax import lax
from jax.experimental import pallas as pl
from jax.experimental.pallas import tpu as pltpu
from jax.experimental.pallas import tpu_sc as plsc

_D = 64
_SEQ = 200
_NC = 2
_NS = 16
_NW = _NC * _NS
_BPW = 128
_NV = _D // 16


def _sc_body(table_hbm, xt_hbm, pe_hbm, out_hbm,
             idx_v, pe_v, rows0, rows1, out0, out1, gsem0, gsem1,
             ssem0, ssem1):
    wid = lax.axis_index("s") * _NC + lax.axis_index("c")
    b0 = wid * _BPW

    pltpu.sync_copy(xt_hbm.at[:, pl.ds(b0, _BPW)], idx_v)
    pltpu.sync_copy(pe_hbm, pe_v)

    def fire(l, rows_b, gsem):
        pltpu.async_copy(table_hbm.at[idx_v.at[l]], rows_b, gsem)

    def wait_gather(rows_b, gsem):
        pltpu.make_async_copy(table_hbm.at[pl.ds(0, _BPW)], rows_b,
                              gsem).wait()

    iota = lax.iota(jnp.int32, 16)
    half = lax.shift_right_logical(iota, 1 + 1 + 1)
    r_sub = [half + 2 * s for s in range(_NV)]
    dr_vec = lax.bitwise_and(iota, 7)

    def compute(l, rows_b, out_b):
        pvec = [pe_v[l, pl.ds(s * 16, 16)] for s in range(_NV)]

        @plsc.parallel_loop(0, _BPW, unroll=4)
        def _(b):
            bc = jnp.broadcast_to(b, (16,))
            for s in range(_NV):
                v = rows_b[b, pl.ds(s * 16, 16)] * 8.0 + pvec[s]
                plsc.store_scatter(out_b, [r_sub[s], dr_vec, bc], v)

    def store(l, out_b, ssem):
        pltpu.async_copy(out_b.at[:, :, pl.ds(0, 128)], out_hbm.at[l, :, wid],
                         ssem)

    def wait_store(out_b, ssem):
        pltpu.make_async_copy(out_b.at[:, :, pl.ds(0, 128)],
                              out_hbm.at[0, :, wid], ssem).wait()

    fire(0, rows0, gsem0)

    def loop_body(t, _):
        l = 2 * t
        fire(l + 1, rows1, gsem1)
        wait_gather(rows0, gsem0)

        @pl.when(t > 0)
        def _():
            wait_store(out0, ssem0)
        compute(l, rows0, out0)
        store(l, out0, ssem0)

        @pl.when(l + 2 < _SEQ)
        def _():
            fire(l + 2, rows0, gsem0)
        wait_gather(rows1, gsem1)

        @pl.when(t > 0)
        def _():
            wait_store(out1, ssem1)
        compute(l + 1, rows1, out1)
        store(l + 1, out1, ssem1)
        return ()

    lax.fori_loop(0, _SEQ // 2, loop_body, ())
    wait_store(out0, ssem0)
    wait_store(out1, ssem1)


def kernel(x, table, pe):
    b, seq = x.shape
    assert seq == _SEQ and b == _NW * _BPW
    xt = x.T.astype(jnp.int32)
    pe2 = pe[0, :seq, :]

    mesh = plsc.VectorSubcoreMesh(core_axis_name="c", subcore_axis_name="s",
                                  num_cores=_NC, num_subcores=_NS)
    grid_kernel = pl.kernel(
        _sc_body,
        out_type=jax.ShapeDtypeStruct((_SEQ, _D // 8, b // 128, 8, 128),
                                      jnp.float32),
        mesh=mesh,
        scratch_types=[
            pltpu.VMEM((_SEQ, _BPW), jnp.int32),
            pltpu.VMEM((_SEQ, _D), jnp.float32),
            pltpu.VMEM((_BPW, _D), jnp.float32),
            pltpu.VMEM((_BPW, _D), jnp.float32),
            pltpu.VMEM((_D // 8, 8, 129), jnp.float32),
            pltpu.VMEM((_D // 8, 8, 129), jnp.float32),
            pltpu.SemaphoreType.DMA,
            pltpu.SemaphoreType.DMA,
            pltpu.SemaphoreType.DMA,
            pltpu.SemaphoreType.DMA,
        ],
        compiler_params=pltpu.CompilerParams(use_tc_tiling_on_sc=False,
                                             needs_layout_passes=False),
    )
    out5 = grid_kernel(table, xt, pe2)
    return out5.transpose(2, 4, 0, 1, 3).reshape(b, seq, _D)
